# baseline (device time: 22101 ns/iter reference)
import jax
import jax.numpy as jnp
from jax import lax
from jax.experimental import pallas as pl
from jax.experimental.pallas import tpu as pltpu

N_DEV = 8
TAPS = 4
HALO = TAPS - 1
SEQ_TILE = 256


def kernel(x, k):
    b, s, c = x.shape
    n_tiles = s // SEQ_TILE

    def body(x_tile_ref, x_full_ref, k_ref, out_ref, carry_ref, send_sem, recv_sem):
        my = lax.axis_index("i")
        j = pl.program_id(0)

        @pl.when(j == 0)
        def _():
            @pl.when(my < N_DEV - 1)
            def _():
                send = pltpu.make_async_remote_copy(
                    src_ref=x_full_ref.at[:, pl.ds(s - HALO, HALO), :],
                    dst_ref=carry_ref,
                    send_sem=send_sem,
                    recv_sem=recv_sem,
                    device_id=(my + 1,),
                    device_id_type=pl.DeviceIdType.MESH,
                )
                send.start()
                send.wait_send()

            @pl.when(my == 0)
            def _():
                carry_ref[...] = jnp.zeros((b, HALO, c), jnp.float32)

            @pl.when(my > 0)
            def _():
                recv = pltpu.make_async_remote_copy(
                    src_ref=x_full_ref.at[:, pl.ds(s - HALO, HALO), :],
                    dst_ref=carry_ref,
                    send_sem=send_sem,
                    recv_sem=recv_sem,
                    device_id=(my - 1,),
                    device_id_type=pl.DeviceIdType.MESH,
                )
                recv.wait_recv()

        xv = x_tile_ref[...]
        hv = carry_ref[...]
        ext = jnp.concatenate([hv, xv], axis=1)
        acc = ext[:, 0:SEQ_TILE, :] * k_ref[0][None, None, :]
        for t in range(1, TAPS):
            acc = acc + ext[:, t:t + SEQ_TILE, :] * k_ref[t][None, None, :]
        out_ref[...] = acc * jax.nn.sigmoid(acc)

        carry_ref[...] = xv[:, SEQ_TILE - HALO:, :]

    return pl.pallas_call(
        body,
        grid=(n_tiles,),
        out_shape=jax.ShapeDtypeStruct((b, s, c), jnp.float32),
        in_specs=[
            pl.BlockSpec((b, SEQ_TILE, c), lambda j: (0, j, 0)),
            pl.BlockSpec(memory_space=pl.ANY),
            pl.BlockSpec((TAPS, c), lambda j: (0, 0)),
        ],
        out_specs=pl.BlockSpec((b, SEQ_TILE, c), lambda j: (0, j, 0)),
        scratch_shapes=[
            pltpu.VMEM((b, HALO, c), jnp.float32),
            pltpu.SemaphoreType.DMA,
            pltpu.SemaphoreType.DMA,
        ],
        compiler_params=pltpu.CompilerParams(
            dimension_semantics=("arbitrary",),
        ),
    )(x, x, k)


# device time: 16710 ns/iter; 1.3226x vs baseline; 1.3226x over previous
import jax
import jax.numpy as jnp
from jax import lax
from jax.experimental import pallas as pl
from jax.experimental.pallas import tpu as pltpu

N_DEV = 8
TAPS = 4
HALO = TAPS - 1
SEQ_TILE = 256


def kernel(x, k):
    b, s, c = x.shape
    n_tiles = s // SEQ_TILE

    def body(x_tile_ref, x_full_ref, k_ref, out_hbm, ybuf, halo_ref, carry_ref,
             fixbuf, send_sem, recv_sem, out_sems, fix_sem):
        my = lax.axis_index("i")
        j = pl.program_id(0)

        def halo_rdma(target):
            return pltpu.make_async_remote_copy(
                src_ref=x_full_ref.at[:, pl.ds(s - HALO, HALO), :],
                dst_ref=halo_ref,
                send_sem=send_sem,
                recv_sem=recv_sem,
                device_id=(target,),
                device_id_type=pl.DeviceIdType.MESH,
            )

        def out_copy(jj, slot):
            return pltpu.make_async_copy(
                ybuf.at[slot],
                out_hbm.at[:, pl.ds(jj * SEQ_TILE, SEQ_TILE), :],
                out_sems.at[slot],
            )

        def conv_silu(hv, xv, n):
            ext = jnp.concatenate([hv, xv], axis=1)
            acc = ext[:, 0:n, :] * k_ref[0][None, None, :]
            for t in range(1, TAPS):
                acc = acc + ext[:, t:t + n, :] * k_ref[t][None, None, :]
            return acc * jax.nn.sigmoid(acc)

        barrier = pltpu.get_barrier_semaphore()

        @pl.when(j == 0)
        def _():
            @pl.when(my > 0)
            def _():
                pl.semaphore_signal(barrier, inc=1, device_id=(my - 1,),
                                    device_id_type=pl.DeviceIdType.MESH)

            carry_ref[...] = jnp.zeros((b, HALO, c), jnp.float32)

        @pl.when(j == 1)
        def _():
            @pl.when(my < N_DEV - 1)
            def _():
                pl.semaphore_wait(barrier, 1)
                halo_rdma(my + 1).start()

        slot = lax.rem(j, 2)

        @pl.when(j >= 2)
        def _():
            out_copy(j - 2, slot).wait()

        xv = x_tile_ref[...]

        @pl.when(j == 0)
        def _():
            fixbuf[1] = xv[:, 0:HALO, :]

        ybuf[slot] = conv_silu(carry_ref[...], xv, SEQ_TILE)
        carry_ref[...] = xv[:, SEQ_TILE - HALO:, :]
        out_copy(j, slot).start()

        @pl.when(j == n_tiles - 1)
        def _():
            out_copy(n_tiles - 2, lax.rem(n_tiles - 2, 2)).wait()
            out_copy(n_tiles - 1, lax.rem(n_tiles - 1, 2)).wait()

            @pl.when(my > 0)
            def _():
                halo_rdma(my - 1).wait_recv()
                fixbuf[0] = conv_silu(halo_ref[...], fixbuf[1], HALO)
                fix_out = pltpu.make_async_copy(
                    fixbuf.at[0],
                    out_hbm.at[:, pl.ds(0, HALO), :],
                    fix_sem,
                )
                fix_out.start()
                fix_out.wait()

            @pl.when(my < N_DEV - 1)
            def _():
                halo_rdma(my + 1).wait_send()

    return pl.pallas_call(
        body,
        grid=(n_tiles,),
        out_shape=jax.ShapeDtypeStruct((b, s, c), jnp.float32),
        in_specs=[
            pl.BlockSpec((b, SEQ_TILE, c), lambda j: (0, j, 0)),
            pl.BlockSpec(memory_space=pl.ANY),
            pl.BlockSpec((TAPS, c), lambda j: (0, 0)),
        ],
        out_specs=pl.BlockSpec(memory_space=pl.ANY),
        scratch_shapes=[
            pltpu.VMEM((2, b, SEQ_TILE, c), jnp.float32),
            pltpu.VMEM((b, HALO, c), jnp.float32),
            pltpu.VMEM((b, HALO, c), jnp.float32),
            pltpu.VMEM((2, b, HALO, c), jnp.float32),
            pltpu.SemaphoreType.DMA,
            pltpu.SemaphoreType.DMA,
            pltpu.SemaphoreType.DMA((2,)),
            pltpu.SemaphoreType.DMA,
        ],
        compiler_params=pltpu.CompilerParams(
            dimension_semantics=("arbitrary",),
            collective_id=0,
        ),
    )(x, x, k)
